# Initial kernel scaffold; baseline (speedup 1.0000x reference)
import jax
import jax.numpy as jnp
from jax.experimental import pallas as pl


def kernel(cat, cont, tables, W, b):
    raise NotImplementedError("write your pallas kernel here")


# speedup vs baseline: 1.0327x; 1.0327x over previous
import functools
import math

import numpy as _np

import jax
import jax.numpy as jnp
from jax import lax
from jax.experimental import pallas as pl
from jax.experimental.pallas import tpu as pltpu
from jax.experimental.pallas import tpu_sc as plsc

_B, _L, _C, _F = 1024, 20, 26, 13
_V, _D = 100000, 32
_FP = 8
_BL = _B * _L
_K = _C + _F

_NC, _NS = 2, 16
_NW = _NC * _NS
_R = _BL * _C
_PR = _R // _NW
_CH = 128
_NCH = _PR // _CH
_SLAB = 128
_NP = _C * _V * _D // _SLAB


def _sc_gather_body(tab_hbm, gidx_hbm, out_hbm, idx_v, buf_v, sem0, sem1):
    wid = lax.axis_index("s") * _NC + lax.axis_index("c")
    row0 = wid * _PR
    pltpu.sync_copy(gidx_hbm.at[wid], idx_v)
    sems = (sem0, sem1)
    for b in range(2):
        pltpu.async_copy(tab_hbm.at[idx_v.at[b]], buf_v.at[b], sems[b])

    @pl.loop(0, _NCH, step=2)
    def _chunks(g):
        for b in range(2):
            c = g + b
            pltpu.make_async_copy(
                tab_hbm.at[idx_v.at[c]], buf_v.at[b], sems[b]).wait()
            pltpu.sync_copy(buf_v.at[b],
                            out_hbm.at[pl.ds(row0 + c * _CH, _CH)])

            @pl.when(c + 2 < _NCH)
            def _():
                pltpu.async_copy(tab_hbm.at[idx_v.at[c + 2]], buf_v.at[b],
                                 sems[b])


def _sc_gather(tabp, gidx2):
    mesh = plsc.VectorSubcoreMesh(core_axis_name="c", subcore_axis_name="s")
    run = pl.kernel(
        _sc_gather_body,
        out_type=jax.ShapeDtypeStruct((_R, _SLAB), jnp.float32),
        mesh=mesh,
        scratch_types=[
            pltpu.VMEM((_NCH, _CH), jnp.int32),
            pltpu.VMEM((2, _CH, _SLAB), jnp.float32),
            pltpu.SemaphoreType.DMA,
            pltpu.SemaphoreType.DMA,
        ],
    )
    return run(tabp, gidx2)


_ROWS = 128


def _tc_body(slab_ref, sub_ref, cont_ref, ws_ref, wc_ref, b_ref, out_ref):
    x = slab_ref[...]
    sub = sub_ref[...]
    acc = jnp.zeros((_ROWS * _C, _D), jnp.float32)
    for k in range(4):
        acc = acc + jnp.where(sub == k, x[:, k * _D:(k + 1) * _D], 0.0)
    out_ref[:, 0:_C, :] = acc.reshape(_ROWS, _C, _D)

    it = lax.broadcasted_iota(jnp.int32, (1, _FP), 1)
    fm = (1 << it).astype(jnp.float32) * math.pi
    cv = cont_ref[...]
    for i in range(_F):
        ang = cv[:, i:i + 1] * fm
        s = jnp.sin(ang)
        c = jnp.cos(ang)
        p = jnp.dot(s, ws_ref[i], preferred_element_type=jnp.float32)
        p = p + jnp.dot(c, wc_ref[i], preferred_element_type=jnp.float32)
        out_ref[:, _C + i, :] = p + b_ref[i][None, :]


def _tc_assemble(slabs, sub, cont2, ws, wc, bias):
    return pl.pallas_call(
        _tc_body,
        out_shape=jax.ShapeDtypeStruct((_BL, _K, _D), jnp.float32),
        grid=(_BL // _ROWS,),
        in_specs=[
            pl.BlockSpec((_ROWS * _C, _SLAB), lambda i: (i, 0)),
            pl.BlockSpec((_ROWS * _C, 1), lambda i: (i, 0)),
            pl.BlockSpec((_ROWS, _F), lambda i: (i, 0)),
            pl.BlockSpec((_F, _FP, _D), lambda i: (0, 0, 0)),
            pl.BlockSpec((_F, _FP, _D), lambda i: (0, 0, 0)),
            pl.BlockSpec((_F, _D), lambda i: (0, 0)),
        ],
        out_specs=pl.BlockSpec((_ROWS, _K, _D), lambda i: (i, 0, 0)),
    )(slabs, sub, cont2, ws, wc, bias)


def kernel(cat, cont, tables, W, b):
    cat2 = cat.reshape(_BL, _C).astype(jnp.int32)
    flat = cat2 + (jnp.arange(_C, dtype=jnp.int32) * _V)[None, :]
    gidx2 = (flat >> 2).reshape(_NW, _NCH, _CH)
    sub = (flat & 3).reshape(_R, 1)

    tabp = tables.reshape(_NP, _SLAB)

    slabs = _sc_gather(tabp, gidx2)

    cont2 = cont.reshape(_BL, _F)
    ws = W[:, 0::2, :]
    wc = W[:, 1::2, :]
    return _tc_assemble(slabs, sub, cont2, ws, wc, b)


# speedup vs baseline: 1.1105x; 1.0754x over previous
import functools
import math

import numpy as _np

import jax
import jax.numpy as jnp
from jax import lax
from jax.experimental import pallas as pl
from jax.experimental.pallas import tpu as pltpu
from jax.experimental.pallas import tpu_sc as plsc

_B, _L, _C, _F = 1024, 20, 26, 13
_V, _D = 100000, 32
_FP = 8
_BL = _B * _L
_K = _C + _F

_NC, _NS = 2, 16
_NW = _NC * _NS
_R = _BL * _C
_PR = _R // _NW
_CH = 128
_NCH = _PR // _CH
_SLAB = 128
_NP = _C * _V * _D // _SLAB


def _sc_gather_body(tab_hbm, gidx_hbm, sub_hbm, out_hbm,
                    idx_v, sub_v, buf_v, obuf_v,
                    semg0, semg1, sems):
    wid = lax.axis_index("s") * _NC + lax.axis_index("c")
    row0 = wid * _PR
    pltpu.sync_copy(gidx_hbm.at[wid], idx_v)
    pltpu.sync_copy(sub_hbm.at[wid], sub_v)
    semg = (semg0, semg1)
    for b in range(2):
        pltpu.async_copy(tab_hbm.at[idx_v.at[b]], buf_v.at[b], semg[b])

    @pl.loop(0, _NCH, step=2)
    def _chunks(g):
        for b in range(2):
            c = g + b
            pltpu.make_async_copy(
                tab_hbm.at[idx_v.at[c]], buf_v.at[b], semg[b]).wait()
            for rg in range(8):
                rows = lax.iota(jnp.int32, 16) + (16 * rg)
                colb = sub_v[c, pl.ds(16 * rg, 16)]
                for j in range(_D):
                    val = plsc.load_gather(buf_v.at[b], [rows, colb + j])
                    plsc.store_scatter(obuf_v, [rows, jnp.full((16,), j, jnp.int32)], val)

            @pl.when(c + 2 < _NCH)
            def _():
                pltpu.async_copy(tab_hbm.at[idx_v.at[c + 2]], buf_v.at[b],
                                 semg[b])

            pltpu.sync_copy(obuf_v,
                            out_hbm.at[pl.ds(row0 + c * _CH, _CH)])


def _sc_gather(tabp, gidx2, sub2):
    mesh = plsc.VectorSubcoreMesh(core_axis_name="c", subcore_axis_name="s")
    run = pl.kernel(
        _sc_gather_body,
        out_type=jax.ShapeDtypeStruct((_R, _D), jnp.float32),
        mesh=mesh,
        compiler_params=pltpu.CompilerParams(needs_layout_passes=False),
        scratch_types=[
            pltpu.VMEM((_NCH, _CH), jnp.int32),
            pltpu.VMEM((_NCH, _CH), jnp.int32),
            pltpu.VMEM((2, _CH, _SLAB), jnp.float32),
            pltpu.VMEM((_CH, _D), jnp.float32),
            pltpu.SemaphoreType.DMA,
            pltpu.SemaphoreType.DMA,
            pltpu.SemaphoreType.DMA,
        ],
    )
    return run(tabp, gidx2, sub2)


_ROWS = 128


def _tc_body(cat_ref, cont_ref, ws_ref, wc_ref, b_ref, out_ref):
    out_ref[:, 0:_C, :] = cat_ref[...].reshape(_ROWS, _C, _D)

    it = lax.broadcasted_iota(jnp.int32, (1, _FP), 1)
    fm = (1 << it).astype(jnp.float32) * math.pi
    cv = cont_ref[...]
    for i in range(_F):
        ang = cv[:, i:i + 1] * fm
        s = jnp.sin(ang)
        c = jnp.cos(ang)
        p = jnp.dot(s, ws_ref[i], preferred_element_type=jnp.float32)
        p = p + jnp.dot(c, wc_ref[i], preferred_element_type=jnp.float32)
        out_ref[:, _C + i, :] = p + b_ref[i][None, :]


def _tc_assemble(catrows, cont2, ws, wc, bias):
    return pl.pallas_call(
        _tc_body,
        out_shape=jax.ShapeDtypeStruct((_BL, _K, _D), jnp.float32),
        grid=(_BL // _ROWS,),
        in_specs=[
            pl.BlockSpec((_ROWS * _C, _D), lambda i: (i, 0)),
            pl.BlockSpec((_ROWS, _F), lambda i: (i, 0)),
            pl.BlockSpec((_F, _FP, _D), lambda i: (0, 0, 0)),
            pl.BlockSpec((_F, _FP, _D), lambda i: (0, 0, 0)),
            pl.BlockSpec((_F, _D), lambda i: (0, 0)),
        ],
        out_specs=pl.BlockSpec((_ROWS, _K, _D), lambda i: (i, 0, 0)),
    )(catrows, cont2, ws, wc, bias)


def kernel(cat, cont, tables, W, b):
    cat2 = cat.reshape(_BL, _C).astype(jnp.int32)
    flat = cat2 + (jnp.arange(_C, dtype=jnp.int32) * _V)[None, :]
    gidx2 = (flat >> 2).reshape(_NW, _NCH, _CH)
    sub2 = ((flat & 3) << 5).reshape(_NW, _NCH, _CH)

    tabp = tables.reshape(_NP, _SLAB)

    catrows = _sc_gather(tabp, gidx2, sub2)

    cont2 = cont.reshape(_BL, _F)
    ws = W[:, 0::2, :]
    wc = W[:, 1::2, :]
    return _tc_assemble(catrows, cont2, ws, wc, b)


# speedup vs baseline: 1.2616x; 1.1360x over previous
import functools
import math

import numpy as _np

import jax
import jax.numpy as jnp
from jax import lax
from jax.experimental import pallas as pl
from jax.experimental.pallas import tpu as pltpu
from jax.experimental.pallas import tpu_sc as plsc

_B, _L, _C, _F = 1024, 20, 26, 13
_V, _D = 100000, 32
_FP = 8
_BL = _B * _L
_K = _C + _F

_NC, _NS = 2, 16
_NW = _NC * _NS
_R = _BL * _C
_PR = _R // _NW
_CH = 128
_NCH = _PR // _CH
_SLAB = 128
_NP = _C * _V * _D // _SLAB


def _sc_gather_body(tab_hbm, gidx_hbm, sub_hbm, out_hbm,
                    idx_v, sub_v, buf_v, obuf_v,
                    semg0, semg1, sems):
    wid = lax.axis_index("s") * _NC + lax.axis_index("c")
    row0 = wid * _PR
    pltpu.sync_copy(gidx_hbm.at[wid], idx_v)
    pltpu.sync_copy(sub_hbm.at[wid], sub_v)
    semg = (semg0, semg1)
    for b in range(2):
        pltpu.async_copy(tab_hbm.at[idx_v.at[b]], buf_v.at[b], semg[b])

    @pl.loop(0, _NCH, step=2)
    def _chunks(g):
        for b in range(2):
            c = g + b
            pltpu.make_async_copy(
                tab_hbm.at[idx_v.at[c]], buf_v.at[b], semg[b]).wait()
            for rg in range(8):
                rows = lax.iota(jnp.int32, 16) + (16 * rg)
                colb = sub_v[c, pl.ds(16 * rg, 16)]
                for j in range(_D):
                    val = plsc.load_gather(buf_v.at[b], [rows, colb + j])
                    plsc.store_scatter(obuf_v, [rows, jnp.full((16,), j, jnp.int32)], val)

            @pl.when(c + 2 < _NCH)
            def _():
                pltpu.async_copy(tab_hbm.at[idx_v.at[c + 2]], buf_v.at[b],
                                 semg[b])

            pltpu.sync_copy(obuf_v,
                            out_hbm.at[pl.ds(row0 + c * _CH, _CH)])


def _sc_gather(tabp, gidx2, sub2):
    mesh = plsc.VectorSubcoreMesh(core_axis_name="c", subcore_axis_name="s")
    run = pl.kernel(
        _sc_gather_body,
        out_type=jax.ShapeDtypeStruct((_R, _D), jnp.float32),
        mesh=mesh,
        compiler_params=pltpu.CompilerParams(needs_layout_passes=False),
        scratch_types=[
            pltpu.VMEM((_NCH, _CH), jnp.int32),
            pltpu.VMEM((_NCH, _CH), jnp.int32),
            pltpu.VMEM((2, _CH, _SLAB), jnp.float32),
            pltpu.VMEM((_CH, _D), jnp.float32),
            pltpu.SemaphoreType.DMA,
            pltpu.SemaphoreType.DMA,
            pltpu.SemaphoreType.DMA,
        ],
    )
    return run(tabp, gidx2, sub2)


_ROWS = 128


_FD = _F * _FP
_FO = _F * _D


def _tc_body(cat_ref, rep_ref, wsbd_ref, wcbd_ref, b_ref, out_ref):
    out_ref[:, 0:_C, :] = cat_ref[...].reshape(_ROWS, _C, _D)

    it = lax.broadcasted_iota(jnp.int32, (1, _FD), 1)
    fm = (1 << (it & 7)).astype(jnp.float32) * math.pi
    ang = rep_ref[...] * fm
    res = jnp.dot(jnp.sin(ang), wsbd_ref[...],
                  preferred_element_type=jnp.float32)
    res = res + jnp.dot(jnp.cos(ang), wcbd_ref[...],
                        preferred_element_type=jnp.float32)
    res = res + b_ref[...]
    for i in range(_F):
        out_ref[:, _C + i, :] = res[:, i * _D:(i + 1) * _D]


def _tc_assemble(catrows, cont_rep, wsbd, wcbd, bias416):
    return pl.pallas_call(
        _tc_body,
        out_shape=jax.ShapeDtypeStruct((_BL, _K, _D), jnp.float32),
        grid=(_BL // _ROWS,),
        in_specs=[
            pl.BlockSpec((_ROWS * _C, _D), lambda i: (i, 0)),
            pl.BlockSpec((_ROWS, _FD), lambda i: (i, 0)),
            pl.BlockSpec((_FD, _FO), lambda i: (0, 0)),
            pl.BlockSpec((_FD, _FO), lambda i: (0, 0)),
            pl.BlockSpec((1, _FO), lambda i: (0, 0)),
        ],
        out_specs=pl.BlockSpec((_ROWS, _K, _D), lambda i: (i, 0, 0)),
    )(catrows, cont_rep, wsbd, wcbd, bias416)


def kernel(cat, cont, tables, W, b):
    cat2 = cat.reshape(_BL, _C).astype(jnp.int32)
    flat = cat2 + (jnp.arange(_C, dtype=jnp.int32) * _V)[None, :]
    gidx2 = (flat >> 2).reshape(_NW, _NCH, _CH)
    sub2 = ((flat & 3) << 5).reshape(_NW, _NCH, _CH)

    tabp = tables.reshape(_NP, _SLAB)

    catrows = _sc_gather(tabp, gidx2, sub2)

    cont_rep = jnp.repeat(cont.reshape(_BL, _F), _FP, axis=1)
    eye = jnp.eye(_F, dtype=jnp.float32)
    ws = W[:, 0::2, :]
    wc = W[:, 1::2, :]
    wsbd = (eye[:, None, :, None] * ws[:, :, None, :]).reshape(_FD, _FO)
    wcbd = (eye[:, None, :, None] * wc[:, :, None, :]).reshape(_FD, _FO)
    bias416 = b.reshape(1, _FO)
    return _tc_assemble(catrows, cont_rep, wsbd, wcbd, bias416)


# speedup vs baseline: 2.1219x; 1.6819x over previous
import functools
import math

import numpy as _np

import jax
import jax.numpy as jnp
from jax import lax
from jax.experimental import pallas as pl
from jax.experimental.pallas import tpu as pltpu
from jax.experimental.pallas import tpu_sc as plsc

_B, _L, _C, _F = 1024, 20, 26, 13
_V, _D = 100000, 32
_FP = 8
_BL = _B * _L
_K = _C + _F

_NC, _NS = 2, 16
_NW = _NC * _NS
_TW = _BL // _NW
_CH = 128
_CPF = _TW // _CH
_NCH = _C * _CPF
_SLAB = 128
_NP = _C * _V * _D // _SLAB


def _sc_gather_body(tab_hbm, gidx_hbm, sub_hbm, out_hbm,
                    idx_v, sub_v, buf_v, obuf_v, sem0, sem1):
    wid = lax.axis_index("s") * _NC + lax.axis_index("c")
    col0 = wid * _TW
    pltpu.sync_copy(gidx_hbm.at[wid], idx_v)
    pltpu.sync_copy(sub_hbm.at[wid], sub_v)
    sems = (sem0, sem1)
    for b in range(2):
        pltpu.async_copy(tab_hbm.at[idx_v.at[b]], buf_v.at[b], sems[b])

    lanes = lax.iota(jnp.int32, 16)

    def do_chunk(k, i, cc, b):
        pltpu.make_async_copy(
            tab_hbm.at[idx_v.at[k]], buf_v.at[b], sems[b]).wait()
        colbs = [sub_v[k, pl.ds(16 * rg, 16)] for rg in range(8)]
        toks = [lanes + (16 * rg) for rg in range(8)]
        for j in range(_D):
            jl = (j + lanes) & 31
            for rg in range(8):
                val = plsc.load_gather(buf_v.at[b], [toks[rg], colbs[rg] + jl])
                plsc.store_scatter(obuf_v, [jl, toks[rg]], val)

        @pl.when(k + 2 < _NCH)
        def _():
            pltpu.async_copy(tab_hbm.at[idx_v.at[k + 2]], buf_v.at[b],
                             sems[b])

        pltpu.sync_copy(
            obuf_v,
            out_hbm.at[pl.ds(i * _D, _D), pl.ds(col0 + cc * _CH, _CH)])

    def step(i, cc):
        wrap = cc == (_CPF - 1)
        return (i + wrap.astype(jnp.int32),
                jnp.where(wrap, 0, cc + 1))

    def body(g, carry):
        i, cc = carry
        do_chunk(2 * g, i, cc, 0)
        i, cc = step(i, cc)
        do_chunk(2 * g + 1, i, cc, 1)
        return step(i, cc)

    lax.fori_loop(0, _NCH // 2, body, (jnp.int32(0), jnp.int32(0)))


def _sc_gather(tabp, gidxT, subT):
    mesh = plsc.VectorSubcoreMesh(core_axis_name="c", subcore_axis_name="s")
    run = pl.kernel(
        _sc_gather_body,
        out_type=jax.ShapeDtypeStruct((_K * _D, _BL), jnp.float32),
        mesh=mesh,
        compiler_params=pltpu.CompilerParams(needs_layout_passes=False),
        scratch_types=[
            pltpu.VMEM((_NCH, _CH), jnp.int32),
            pltpu.VMEM((_NCH, _CH), jnp.int32),
            pltpu.VMEM((2, _CH, _SLAB), jnp.float32),
            pltpu.VMEM((_D, _CH), jnp.float32),
            pltpu.SemaphoreType.DMA,
            pltpu.SemaphoreType.DMA,
        ],
    )
    return run(tabp, gidxT, subT)


_FD = _F * _FP
_FO = _F * _D
_TB = 2560


def _tc_body(out_alias_ref, rep_ref, wsbd_ref, wcbd_ref, b_ref, out_ref):
    del out_alias_ref
    it = lax.broadcasted_iota(jnp.int32, (_FD, 1), 0)
    fm = (1 << (it & 7)).astype(jnp.float32) * math.pi
    ang = rep_ref[...] * fm
    res = jnp.dot(wsbd_ref[...], jnp.sin(ang),
                  preferred_element_type=jnp.float32)
    res = res + jnp.dot(wcbd_ref[...], jnp.cos(ang),
                        preferred_element_type=jnp.float32)
    out_ref[...] = res + b_ref[...]


def _tc_cont(out2, repT, wsbdT, wcbdT, biasT):
    return pl.pallas_call(
        _tc_body,
        out_shape=jax.ShapeDtypeStruct((_K * _D, _BL), jnp.float32),
        grid=(_BL // _TB,),
        in_specs=[
            pl.BlockSpec(memory_space=pl.ANY),
            pl.BlockSpec((_FD, _TB), lambda j: (0, j)),
            pl.BlockSpec((_FO, _FD), lambda j: (0, 0)),
            pl.BlockSpec((_FO, _FD), lambda j: (0, 0)),
            pl.BlockSpec((_FO, 1), lambda j: (0, 0)),
        ],
        out_specs=pl.BlockSpec((_FO, _TB), lambda j: (2, j)),
        input_output_aliases={0: 0},
    )(out2, repT, wsbdT, wcbdT, biasT)


def kernel(cat, cont, tables, W, b):
    cat2 = cat.reshape(_BL, _C).astype(jnp.int32)
    flatT = cat2.T + (jnp.arange(_C, dtype=jnp.int32) * _V)[:, None]
    chunks = flatT.reshape(_C, _NW, _CPF, _CH).transpose(1, 0, 2, 3)
    chunks = chunks.reshape(_NW, _NCH, _CH)
    gidxT = chunks >> 2
    subT = (chunks & 3) << 5

    tabp = tables.reshape(_NP, _SLAB)

    out2 = _sc_gather(tabp, gidxT, subT)

    contT = cont.reshape(_BL, _F).T
    repT = jnp.repeat(contT, _FP, axis=0)
    eye = jnp.eye(_F, dtype=jnp.float32)
    ws = W[:, 0::2, :]
    wc = W[:, 1::2, :]
    wsbdT = (eye[:, None, :, None] * ws[:, :, None, :]).reshape(_FD, _FO).T
    wcbdT = (eye[:, None, :, None] * wc[:, :, None, :]).reshape(_FD, _FO).T
    biasT = b.reshape(1, _FO).T

    full = _tc_cont(out2, repT, wsbdT, wcbdT, biasT)
    return full.reshape(_K, _D, _BL).transpose(2, 0, 1)


# speedup vs baseline: 2.1356x; 1.0065x over previous
import functools
import math

import numpy as _np

import jax
import jax.numpy as jnp
from jax import lax
from jax.experimental import pallas as pl
from jax.experimental.pallas import tpu as pltpu
from jax.experimental.pallas import tpu_sc as plsc

_B, _L, _C, _F = 1024, 20, 26, 13
_V, _D = 100000, 32
_FP = 8
_BL = _B * _L
_K = _C + _F

_NC, _NS = 2, 16
_NW = _NC * _NS
_TW = _BL // _NW
_CH = 128
_CPF = _TW // _CH
_NCH = _C * _CPF
_SLAB = 128
_NP = _C * _V * _D // _SLAB


def _sc_gather_body(tab_hbm, gidx_hbm, sub_hbm, out_hbm,
                    idx_v, sub_v, buf_v, obuf_v, sem0, sem1, semw0, semw1):
    wid = lax.axis_index("s") * _NC + lax.axis_index("c")
    col0 = wid * _TW
    pltpu.sync_copy(gidx_hbm.at[wid], idx_v)
    pltpu.sync_copy(sub_hbm.at[wid], sub_v)
    sems = (sem0, sem1)
    semw = (semw0, semw1)
    for b in range(2):
        pltpu.async_copy(tab_hbm.at[idx_v.at[b]], buf_v.at[b], sems[b])

    lanes = lax.iota(jnp.int32, 16)

    def oslice(i, cc):
        return out_hbm.at[pl.ds(i * _D, _D), pl.ds(col0 + cc * _CH, _CH)]

    def do_chunk(k, i, cc, b, wi, wcc):
        pltpu.make_async_copy(
            tab_hbm.at[idx_v.at[k]], buf_v.at[b], sems[b]).wait()

        @pl.when(k >= 2)
        def _():
            pltpu.make_async_copy(obuf_v.at[b], oslice(wi, wcc),
                                  semw[b]).wait()

        colbs = [sub_v[k, pl.ds(16 * rg, 16)] for rg in range(8)]
        toks = [lanes + (16 * rg) for rg in range(8)]
        for j in range(_D):
            jl = (j + lanes) & 31
            for rg in range(8):
                val = plsc.load_gather(buf_v.at[b], [toks[rg], colbs[rg] + jl])
                plsc.store_scatter(obuf_v.at[b], [jl, toks[rg]], val)

        @pl.when(k + 2 < _NCH)
        def _():
            pltpu.async_copy(tab_hbm.at[idx_v.at[k + 2]], buf_v.at[b],
                             sems[b])

        pltpu.async_copy(obuf_v.at[b], oslice(i, cc), semw[b])

    def step(i, cc):
        wrap = cc == (_CPF - 1)
        return (i + wrap.astype(jnp.int32),
                jnp.where(wrap, 0, cc + 1))

    def body(g, carry):
        i, cc, w0i, w0cc, w1i, w1cc = carry
        do_chunk(2 * g, i, cc, 0, w0i, w0cc)
        n_w0i, n_w0cc = i, cc
        i, cc = step(i, cc)
        do_chunk(2 * g + 1, i, cc, 1, w1i, w1cc)
        n_w1i, n_w1cc = i, cc
        i, cc = step(i, cc)
        return (i, cc, n_w0i, n_w0cc, n_w1i, n_w1cc)

    z = jnp.int32(0)
    lax.fori_loop(0, _NCH // 2, body, (z, z, z, z, z, z))
    pltpu.make_async_copy(obuf_v.at[0], oslice(_C - 1, _CPF - 2),
                          semw[0]).wait()
    pltpu.make_async_copy(obuf_v.at[1], oslice(_C - 1, _CPF - 1),
                          semw[1]).wait()


def _sc_gather(tabp, gidxT, subT):
    mesh = plsc.VectorSubcoreMesh(core_axis_name="c", subcore_axis_name="s")
    run = pl.kernel(
        _sc_gather_body,
        out_type=jax.ShapeDtypeStruct((_K * _D, _BL), jnp.float32),
        mesh=mesh,
        compiler_params=pltpu.CompilerParams(needs_layout_passes=False),
        scratch_types=[
            pltpu.VMEM((_NCH, _CH), jnp.int32),
            pltpu.VMEM((_NCH, _CH), jnp.int32),
            pltpu.VMEM((2, _CH, _SLAB), jnp.float32),
            pltpu.VMEM((2, _D, _CH), jnp.float32),
            pltpu.SemaphoreType.DMA,
            pltpu.SemaphoreType.DMA,
            pltpu.SemaphoreType.DMA,
            pltpu.SemaphoreType.DMA,
        ],
    )
    return run(tabp, gidxT, subT)


_FD = _F * _FP
_FO = _F * _D
_TB = 2560


def _tc_body(out_alias_ref, rep_ref, wsbd_ref, wcbd_ref, b_ref, out_ref):
    del out_alias_ref
    it = lax.broadcasted_iota(jnp.int32, (_FD, 1), 0)
    fm = (1 << (it & 7)).astype(jnp.float32) * math.pi
    ang = rep_ref[...] * fm
    res = jnp.dot(wsbd_ref[...], jnp.sin(ang),
                  preferred_element_type=jnp.float32)
    res = res + jnp.dot(wcbd_ref[...], jnp.cos(ang),
                        preferred_element_type=jnp.float32)
    out_ref[...] = res + b_ref[...]


def _tc_cont(out2, repT, wsbdT, wcbdT, biasT):
    return pl.pallas_call(
        _tc_body,
        out_shape=jax.ShapeDtypeStruct((_K * _D, _BL), jnp.float32),
        grid=(_BL // _TB,),
        in_specs=[
            pl.BlockSpec(memory_space=pl.ANY),
            pl.BlockSpec((_FD, _TB), lambda j: (0, j)),
            pl.BlockSpec((_FO, _FD), lambda j: (0, 0)),
            pl.BlockSpec((_FO, _FD), lambda j: (0, 0)),
            pl.BlockSpec((_FO, 1), lambda j: (0, 0)),
        ],
        out_specs=pl.BlockSpec((_FO, _TB), lambda j: (2, j)),
        input_output_aliases={0: 0},
    )(out2, repT, wsbdT, wcbdT, biasT)


def kernel(cat, cont, tables, W, b):
    cat2 = cat.reshape(_BL, _C).astype(jnp.int32)
    flatT = cat2.T + (jnp.arange(_C, dtype=jnp.int32) * _V)[:, None]
    chunks = flatT.reshape(_C, _NW, _CPF, _CH).transpose(1, 0, 2, 3)
    chunks = chunks.reshape(_NW, _NCH, _CH)
    gidxT = chunks >> 2
    subT = (chunks & 3) << 5

    tabp = tables.reshape(_NP, _SLAB)

    out2 = _sc_gather(tabp, gidxT, subT)

    contT = cont.reshape(_BL, _F).T
    repT = jnp.repeat(contT, _FP, axis=0)
    eye = jnp.eye(_F, dtype=jnp.float32)
    ws = W[:, 0::2, :]
    wc = W[:, 1::2, :]
    wsbdT = (eye[:, None, :, None] * ws[:, :, None, :]).reshape(_FD, _FO).T
    wcbdT = (eye[:, None, :, None] * wc[:, :, None, :]).reshape(_FD, _FO).T
    biasT = b.reshape(1, _FO).T

    full = _tc_cont(out2, repT, wsbdT, wcbdT, biasT)
    return full.reshape(_K, _D, _BL).transpose(2, 0, 1)
